# baseline (speedup 1.0000x reference)
import functools

import jax
import jax.numpy as jnp
from jax import lax
from jax.experimental import pallas as pl
from jax.experimental.pallas import tpu as pltpu
from jax.experimental.pallas import tpu_sc as plsc

VOCAB, D, H, C = 1000000, 32, 32, 10
B, L = 4096, 50

NUM_CORES = 2
NUM_SUBCORES = 16
NW = NUM_CORES * NUM_SUBCORES
SPW = B // NW
LP = 56
SR = 128
VSUP = VOCAB * D // SR
K = 4
NR = SPW // K

_mesh = plsc.VectorSubcoreMesh(core_axis_name="c", subcore_axis_name="s")


@functools.partial(
    pl.kernel,
    mesh=_mesh,
    out_type=jax.ShapeDtypeStruct((B, D), jnp.float32),
    scratch_types=[
        pltpu.VMEM((SPW, LP), jnp.int32),
        pltpu.VMEM((SPW, LP), jnp.int32),
        pltpu.VMEM((K, LP, SR), jnp.float32),
        pltpu.VMEM((K, LP, SR), jnp.float32),
        pltpu.VMEM((K, D), jnp.float32),
        pltpu.VMEM((K, D), jnp.float32),
        pltpu.SemaphoreType.DMA,
        pltpu.SemaphoreType.DMA,
        pltpu.SemaphoreType.DMA,
        pltpu.SemaphoreType.DMA,
    ],
)
def _sc_gather_sum(xsup_hbm, qoff_hbm, table_hbm, out_hbm,
                   idx_v, qoff_v, rows_a, rows_b, out_a, out_b,
                   sem_a, sem_b, sem_oa, sem_ob):
    wid = lax.axis_index("s") * NUM_CORES + lax.axis_index("c")
    base = wid * SPW
    pltpu.sync_copy(xsup_hbm.at[pl.ds(base, SPW)], idx_v)
    pltpu.sync_copy(qoff_hbm.at[pl.ds(base, SPW)], qoff_v)

    def issue(buf, sem, r):
        @pl.when(r < NR)
        def _():
            for j in range(K):
                pltpu.async_copy(
                    table_hbm.at[idx_v.at[r * K + j]], buf.at[j], sem)

    def drain(buf, sem):
        for j in range(K):
            pltpu.make_async_copy(
                table_hbm.at[idx_v.at[0]], buf.at[j], sem).wait()

    def consume(buf, out_buf, r):
        for j in range(K):
            s = r * K + j
            qc = [qoff_v[s, pl.ds(0, 16)], qoff_v[s, pl.ds(16, 16)],
                  qoff_v[s, pl.ds(32, 16)], qoff_v[s, pl.ds(40, 16)]]

            def qat(t):
                if t < 48:
                    return qc[t // 16][t % 16]
                return qc[3][t - 40]

            acc0 = buf[j, 0, pl.ds(qat(0), 16)]
            acc1 = buf[j, 0, pl.ds(qat(0) + 16, 16)]
            for t in range(1, L):
                q = qat(t)
                acc0 = acc0 + buf[j, t, pl.ds(q, 16)]
                acc1 = acc1 + buf[j, t, pl.ds(q + 16, 16)]
            out_buf[j, pl.ds(0, 16)] = acc0
            out_buf[j, pl.ds(16, 16)] = acc1

    def store(out_buf, sem_o, r):
        pltpu.async_copy(out_buf, out_hbm.at[pl.ds(base + r * K, K)], sem_o)

    def wait_store(out_buf, sem_o):
        pltpu.make_async_copy(
            out_buf, out_hbm.at[pl.ds(base, K)], sem_o).wait()

    issue(rows_a, sem_a, 0)
    issue(rows_b, sem_b, 1)

    def body(g, _):
        ra = 2 * g
        rb = 2 * g + 1

        @pl.when(g > 0)
        def _():
            wait_store(out_a, sem_oa)
        drain(rows_a, sem_a)
        consume(rows_a, out_a, ra)
        issue(rows_a, sem_a, ra + 2)
        store(out_a, sem_oa, ra)

        @pl.when(g > 0)
        def _():
            wait_store(out_b, sem_ob)
        drain(rows_b, sem_b)
        consume(rows_b, out_b, rb)
        issue(rows_b, sem_b, rb + 2)
        store(out_b, sem_ob, rb)
        return 0

    lax.fori_loop(0, NR // 2, body, 0)
    wait_store(out_a, sem_oa)
    wait_store(out_b, sem_ob)


def _mlp_body(s_ref, l_ref, w1_ref, b1_ref, w2_ref, b2_ref, o_ref):
    rep = s_ref[...] * l_ref[...]
    h = lax.dot_general(rep, w1_ref[...], (((1,), (1,)), ((), ())),
                        preferred_element_type=jnp.float32) + b1_ref[...]
    h = jnp.maximum(h, 0.0)
    o_ref[...] = lax.dot_general(h, w2_ref[...], (((1,), (1,)), ((), ())),
                                 preferred_element_type=jnp.float32) + b2_ref[...]


@jax.jit
def kernel(x, lengths, table, W1, b1, W2, b2):
    xp = jnp.pad(x, ((0, 0), (0, LP - L)))
    xsup = xp >> 2
    qoff = (xp & 3) * D
    table2 = table.reshape(VSUP, SR)
    sums = _sc_gather_sum(xsup, qoff, table2)
    inv_len = (1.0 / lengths.astype(jnp.float32)).reshape(B, 1)
    logits = pl.pallas_call(
        _mlp_body,
        out_shape=jax.ShapeDtypeStruct((B, C), jnp.float32),
    )(sums, inv_len, W1, b1.reshape(1, H), W2, b2.reshape(1, C))
    return logits
